# baseline (speedup 1.0000x reference)
import functools

import jax
import jax.numpy as jnp
from jax import lax
from jax.experimental import pallas as pl
from jax.experimental.pallas import tpu as pltpu
from jax.experimental.pallas import tpu_sc as plsc

N_NODES = 10000
D_FEAT = 256
D_HID = 32
DH2 = 2 * D_HID

NC = 2
NS = 16
NW = NC * NS
EDGES_PER_TILE = 5000
CHUNK = 512
NFULL = 9
TAIL = EDGES_PER_TILE - NFULL * CHUNK
NPAD = 10240
ROWS_PER_TILE = NPAD // NS

_SC_MESH = dict(core_axis_name="c", subcore_axis_name="s",
                num_cores=NC, num_subcores=NS)
_SC_PARAMS = pltpu.CompilerParams(use_tc_tiling_on_sc=False)


def _deg_body(dst_hbm, zeros_hbm, ones_hbm, out_hbm, idx_v, ones_v, acc_sh):
    cid = lax.axis_index("c")
    sid = lax.axis_index("s")
    wid = cid * NS + sid
    row0 = sid * ROWS_PER_TILE
    pltpu.sync_copy(zeros_hbm.at[pl.ds(wid * ROWS_PER_TILE, ROWS_PER_TILE)],
                    acc_sh.at[pl.ds(row0, ROWS_PER_TILE)])
    pltpu.sync_copy(ones_hbm.at[pl.ds(wid * EDGES_PER_TILE, EDGES_PER_TILE)],
                    ones_v)
    pltpu.sync_copy(dst_hbm.at[pl.ds(wid * EDGES_PER_TILE, EDGES_PER_TILE)],
                    idx_v)
    plsc.subcore_barrier()
    pltpu.sync_copy(ones_v, acc_sh.at[idx_v], add=True)
    plsc.subcore_barrier()
    pltpu.sync_copy(acc_sh.at[pl.ds(row0, ROWS_PER_TILE)],
                    out_hbm.at[pl.ds(cid * NPAD + row0, ROWS_PER_TILE)])


_deg_kernel = functools.partial(
    pl.kernel,
    out_type=jax.ShapeDtypeStruct((NC * NPAD,), jnp.float32),
    mesh=plsc.VectorSubcoreMesh(**_SC_MESH),
    scratch_types=[
        pltpu.VMEM((EDGES_PER_TILE,), jnp.int32),
        pltpu.VMEM((EDGES_PER_TILE,), jnp.float32),
        pltpu.VMEM_SHARED((NPAD,), jnp.float32),
    ],
)(_deg_body)


_SIZES = [CHUNK] * NFULL + [TAIL]
_NCH = NFULL + 1


def _agg_body(y_hbm, src_hbm, dst_hbm, out_hbm,
              s0_v, s1_v, dst_v, st_v, dt_v, r0_v, r1_v, acc_sh, sem):
    cid = lax.axis_index("c")
    sid = lax.axis_index("s")
    wid = cid * NS + sid
    row0 = sid * ROWS_PER_TILE
    ebase = wid * EDGES_PER_TILE
    sbufs = (s0_v, s1_v)
    rbufs = (r0_v, r1_v)

    def src_ref(j):
        return st_v if _SIZES[j] == TAIL else sbufs[j % 2]

    def load_src(j):
        pltpu.sync_copy(src_hbm.at[pl.ds(ebase + j * CHUNK, _SIZES[j])],
                        src_ref(j))

    def start_gather(j):
        rows = rbufs[j % 2]
        if _SIZES[j] != CHUNK:
            rows = rows.at[pl.ds(0, _SIZES[j])]
        return pltpu.async_copy(y_hbm.at[src_ref(j)], rows, sem)

    load_src(0)
    cp = start_gather(0)
    pltpu.sync_copy(y_hbm.at[pl.ds(row0, ROWS_PER_TILE)],
                    acc_sh.at[pl.ds(row0, ROWS_PER_TILE)])
    plsc.subcore_barrier()
    for j in range(_NCH):
        if j + 1 < _NCH:
            load_src(j + 1)
        dref = dt_v if _SIZES[j] == TAIL else dst_v
        pltpu.sync_copy(dst_hbm.at[pl.ds(ebase + j * CHUNK, _SIZES[j])], dref)
        cp.wait()
        if j + 1 < _NCH:
            cp = start_gather(j + 1)
        rows = rbufs[j % 2]
        if _SIZES[j] != CHUNK:
            rows = rows.at[pl.ds(0, _SIZES[j])]
        pltpu.sync_copy(rows, acc_sh.at[dref], add=True)
    plsc.subcore_barrier()
    pltpu.sync_copy(acc_sh.at[pl.ds(row0, ROWS_PER_TILE)],
                    out_hbm.at[cid, pl.ds(row0, ROWS_PER_TILE)])


_agg_kernel = functools.partial(
    pl.kernel,
    out_type=jax.ShapeDtypeStruct((NC, NPAD, DH2), jnp.float32),
    mesh=plsc.VectorSubcoreMesh(**_SC_MESH),
    compiler_params=_SC_PARAMS,
    scratch_types=[
        pltpu.VMEM((CHUNK,), jnp.int32),
        pltpu.VMEM((CHUNK,), jnp.int32),
        pltpu.VMEM((CHUNK,), jnp.int32),
        pltpu.VMEM((TAIL,), jnp.int32),
        pltpu.VMEM((TAIL,), jnp.int32),
        pltpu.VMEM((CHUNK, DH2), jnp.float32),
        pltpu.VMEM((CHUNK, DH2), jnp.float32),
        pltpu.VMEM_SHARED((NPAD, DH2), jnp.float32),
        pltpu.SemaphoreType.DMA,
    ],
)(_agg_body)


_ROWS_BLK = 1024
_N_BLKS = NPAD // _ROWS_BLK


def _xw_body(x_ref, w_ref, xw_ref):
    xw_ref[...] = jnp.dot(x_ref[...], w_ref[...],
                          preferred_element_type=jnp.float32)


_XW_BLK = 512
_XW_N = (NPAD // 2) // _XW_BLK


def _xw(x3, wd):
    return pl.pallas_call(
        _xw_body,
        grid=(_XW_N,),
        in_specs=[
            pl.BlockSpec((_XW_BLK, 2 * D_FEAT), lambda i: (i, 0)),
            pl.BlockSpec((2 * D_FEAT, 2 * DH2), lambda i: (0, 0)),
        ],
        out_specs=pl.BlockSpec((_XW_BLK, 2 * DH2), lambda i: (i, 0)),
        out_shape=jax.ShapeDtypeStruct((NPAD // 2, 2 * DH2), jnp.float32),
    )(x3, wd)


_SC_BLK = 1024
_SC_N = (NPAD // 2) // _SC_BLK


def _scale_body(xw_ref, dege_ref, dego_ref, y_ref):
    i = pl.program_id(0)
    dise = lax.rsqrt(dege_ref[...] + 1.0)
    diso = lax.rsqrt(dego_ref[...] + 1.0)
    rows = jax.lax.broadcasted_iota(jnp.int32, (_SC_BLK, 1), 0) + i * _SC_BLK
    ok = rows < (N_NODES // 2)
    y_ref[:, :DH2] = jnp.where(ok, xw_ref[:, :DH2] * dise[:, None], 0.0)
    y_ref[:, DH2:] = jnp.where(ok, xw_ref[:, DH2:] * diso[:, None], 0.0)


def _scale(xwp, dege, dego):
    return pl.pallas_call(
        _scale_body,
        grid=(_SC_N,),
        in_specs=[
            pl.BlockSpec((_SC_BLK, 2 * DH2), lambda i: (i, 0)),
            pl.BlockSpec((_SC_BLK,), lambda i: (i,)),
            pl.BlockSpec((_SC_BLK,), lambda i: (i,)),
        ],
        out_specs=pl.BlockSpec((_SC_BLK, 2 * DH2), lambda i: (i, 0)),
        out_shape=jax.ShapeDtypeStruct((NPAD // 2, 2 * DH2), jnp.float32),
    )(xwp, dege, dego)


_FIN_BLK = 1024
_FIN_N = (NPAD // 2) // _FIN_BLK


def _fin_body(acc_ref, y_ref, dege_ref, dego_ref, m1_ref, m2_ref,
              bz_ref, bh_ref, wl_ref, bl_ref, oute_ref, outo_ref):
    p = acc_ref[0] + acc_ref[1] - y_ref[...]
    dise = lax.rsqrt(dege_ref[...] + 1.0)
    diso = lax.rsqrt(dego_ref[...] + 1.0)
    for half, dis, out_ref in ((0, dise, oute_ref), (1, diso, outo_ref)):
        t = p[:, half * DH2:(half + 1) * DH2] * dis[:, None]
        zin = jnp.dot(t, m1_ref[...], preferred_element_type=jnp.float32) + bz_ref[...]
        hin = jnp.dot(t, m2_ref[...], preferred_element_type=jnp.float32) + bh_ref[...]
        f = (1.0 - jax.nn.sigmoid(zin)) * jnp.tanh(hin)
        out_ref[...] = jnp.sum(f * wl_ref[...], axis=1) + bl_ref[0, 0]


def _final(acc_p, y_p, dege, dego, m1, m2, bz, bh, wlrow, blin):
    return pl.pallas_call(
        _fin_body,
        grid=(_FIN_N,),
        in_specs=[
            pl.BlockSpec((NC, _FIN_BLK, 2 * DH2), lambda i: (0, i, 0)),
            pl.BlockSpec((_FIN_BLK, 2 * DH2), lambda i: (i, 0)),
            pl.BlockSpec((_FIN_BLK,), lambda i: (i,)),
            pl.BlockSpec((_FIN_BLK,), lambda i: (i,)),
            pl.BlockSpec((DH2, D_HID), lambda i: (0, 0)),
            pl.BlockSpec((DH2, D_HID), lambda i: (0, 0)),
            pl.BlockSpec((1, D_HID), lambda i: (0, 0)),
            pl.BlockSpec((1, D_HID), lambda i: (0, 0)),
            pl.BlockSpec((1, D_HID), lambda i: (0, 0)),
            pl.BlockSpec((1, 1), lambda i: (0, 0)),
        ],
        out_specs=(pl.BlockSpec((_FIN_BLK,), lambda i: (i,)),
                   pl.BlockSpec((_FIN_BLK,), lambda i: (i,))),
        out_shape=(jax.ShapeDtypeStruct((NPAD // 2,), jnp.float32),
                   jax.ShapeDtypeStruct((NPAD // 2,), jnp.float32)),
    )(acc_p, y_p, dege, dego, m1, m2, bz, bh, wlrow, blin)


def kernel(x, edge_index, W_z, b_z, W_r, b_r, W_h, b_h, lz_W, lz_b,
           lr_W, lr_b, lh_W, lh_b, att, W_lin, b_lin):
    f32 = jnp.float32
    x2 = x[:, :, 0]
    wcat = jnp.concatenate([W_z, W_h], axis=1)

    src_flat = edge_index[0]
    dst_flat = edge_index[1]

    zeros_init = jnp.zeros((NW * ROWS_PER_TILE,), f32)
    ones_vals = jnp.ones((NW * EDGES_PER_TILE,), f32)

    x3 = x2.reshape(N_NODES // 2, 2 * D_FEAT)
    zeros_w = jnp.zeros((D_FEAT, DH2), f32)
    wd = jnp.concatenate(
        [jnp.concatenate([wcat, zeros_w], axis=1),
         jnp.concatenate([zeros_w, wcat], axis=1)], axis=0)
    xwp = _xw(x3, wd)
    deg_flat = _deg_kernel(dst_flat, zeros_init, ones_vals)
    deg = deg_flat[:NPAD] + deg_flat[NPAD:]
    dege = deg[0::2]
    dego = deg[1::2]
    y_p = _scale(xwp, dege, dego)
    y_sc = y_p.reshape(NPAD, DH2)
    acc = _agg_kernel(y_sc, src_flat, dst_flat)
    acc_p = acc.reshape(NC, NPAD // 2, 2 * DH2)

    zeros32 = jnp.zeros((D_HID, D_HID), f32)
    m1 = jnp.concatenate([lz_W[:D_HID], zeros32], axis=0)
    m2 = jnp.concatenate([zeros32, lh_W[:D_HID]], axis=0)
    bz = (b_z @ lz_W[:D_HID] + lz_b).reshape(1, D_HID)
    bh = (b_h @ lh_W[:D_HID] + lh_b).reshape(1, D_HID)
    wlrow = W_lin[:, 0].reshape(1, D_HID)
    blin = b_lin.reshape(1, 1)

    oute, outo = _final(acc_p, y_p, dege, dego, m1, m2, bz, bh, wlrow, blin)
    out = jnp.stack([oute, outo], axis=1).reshape(-1)
    return out[:N_NODES]


# speedup vs baseline: 1.5616x; 1.5616x over previous
import functools

import jax
import jax.numpy as jnp
from jax import lax
from jax.experimental import pallas as pl
from jax.experimental.pallas import tpu as pltpu
from jax.experimental.pallas import tpu_sc as plsc

N_NODES = 10000
D_FEAT = 256
D_HID = 32
DH2 = 2 * D_HID

NC = 2
NS = 16
NW = NC * NS
EDGES_PER_TILE = 5000
CHUNK = 512
NFULL = 9
TAIL = EDGES_PER_TILE - NFULL * CHUNK
NPAD = 10240
ROWS_PER_TILE = NPAD // NS

_SC_MESH = dict(core_axis_name="c", subcore_axis_name="s",
                num_cores=NC, num_subcores=NS)
_SC_PARAMS = pltpu.CompilerParams(use_tc_tiling_on_sc=False)


def _deg_body(dst_hbm, zeros_hbm, ones_hbm, out_hbm, idx_v, ones_v, acc_sh):
    cid = lax.axis_index("c")
    sid = lax.axis_index("s")
    wid = cid * NS + sid
    row0 = sid * ROWS_PER_TILE
    pltpu.sync_copy(zeros_hbm.at[pl.ds(wid * ROWS_PER_TILE, ROWS_PER_TILE)],
                    acc_sh.at[pl.ds(row0, ROWS_PER_TILE)])
    pltpu.sync_copy(ones_hbm.at[pl.ds(wid * EDGES_PER_TILE, EDGES_PER_TILE)],
                    ones_v)
    pltpu.sync_copy(dst_hbm.at[pl.ds(wid * EDGES_PER_TILE, EDGES_PER_TILE)],
                    idx_v)
    plsc.subcore_barrier()
    pltpu.sync_copy(ones_v, acc_sh.at[idx_v], add=True)
    plsc.subcore_barrier()
    pltpu.sync_copy(acc_sh.at[pl.ds(row0, ROWS_PER_TILE)],
                    out_hbm.at[pl.ds(cid * NPAD + row0, ROWS_PER_TILE)])


_deg_kernel = functools.partial(
    pl.kernel,
    out_type=jax.ShapeDtypeStruct((NC * NPAD,), jnp.float32),
    mesh=plsc.VectorSubcoreMesh(**_SC_MESH),
    scratch_types=[
        pltpu.VMEM((EDGES_PER_TILE,), jnp.int32),
        pltpu.VMEM((EDGES_PER_TILE,), jnp.float32),
        pltpu.VMEM_SHARED((NPAD,), jnp.float32),
    ],
)(_deg_body)


_SIZES = [CHUNK] * NFULL + [TAIL]
_NCH = NFULL + 1


def _agg_body(y_hbm, src_hbm, dst_hbm, out_hbm,
              s0_v, s1_v, dst_v, st_v, dt_v, r0_v, r1_v, acc_sh, sem):
    cid = lax.axis_index("c")
    sid = lax.axis_index("s")
    wid = cid * NS + sid
    row0 = sid * ROWS_PER_TILE
    ebase = wid * EDGES_PER_TILE
    sbufs = (s0_v, s1_v)
    rbufs = (r0_v, r1_v)

    def src_ref(j):
        return st_v if _SIZES[j] == TAIL else sbufs[j % 2]

    def load_src(j):
        pltpu.sync_copy(src_hbm.at[pl.ds(ebase + j * CHUNK, _SIZES[j])],
                        src_ref(j))

    def start_gather(j):
        rows = rbufs[j % 2]
        if _SIZES[j] != CHUNK:
            rows = rows.at[pl.ds(0, _SIZES[j])]
        return pltpu.async_copy(y_hbm.at[src_ref(j)], rows, sem)

    load_src(0)
    cp = start_gather(0)
    pltpu.sync_copy(y_hbm.at[pl.ds(row0, ROWS_PER_TILE)],
                    acc_sh.at[pl.ds(row0, ROWS_PER_TILE)])
    plsc.subcore_barrier()
    for j in range(_NCH):
        if j + 1 < _NCH:
            load_src(j + 1)
        dref = dt_v if _SIZES[j] == TAIL else dst_v
        pltpu.sync_copy(dst_hbm.at[pl.ds(ebase + j * CHUNK, _SIZES[j])], dref)
        cp.wait()
        if j + 1 < _NCH:
            cp = start_gather(j + 1)
        rows = rbufs[j % 2]
        if _SIZES[j] != CHUNK:
            rows = rows.at[pl.ds(0, _SIZES[j])]
        pltpu.sync_copy(rows, acc_sh.at[dref], add=True)
    plsc.subcore_barrier()
    pltpu.sync_copy(acc_sh.at[pl.ds(row0, ROWS_PER_TILE)],
                    out_hbm.at[cid, pl.ds(row0, ROWS_PER_TILE)])


_agg_kernel = functools.partial(
    pl.kernel,
    out_type=jax.ShapeDtypeStruct((NC, NPAD, DH2), jnp.float32),
    mesh=plsc.VectorSubcoreMesh(**_SC_MESH),
    compiler_params=_SC_PARAMS,
    scratch_types=[
        pltpu.VMEM((CHUNK,), jnp.int32),
        pltpu.VMEM((CHUNK,), jnp.int32),
        pltpu.VMEM((CHUNK,), jnp.int32),
        pltpu.VMEM((TAIL,), jnp.int32),
        pltpu.VMEM((TAIL,), jnp.int32),
        pltpu.VMEM((CHUNK, DH2), jnp.float32),
        pltpu.VMEM((CHUNK, DH2), jnp.float32),
        pltpu.VMEM_SHARED((NPAD, DH2), jnp.float32),
        pltpu.SemaphoreType.DMA,
    ],
)(_agg_body)


_ROWS_BLK = 1024
_N_BLKS = NPAD // _ROWS_BLK


def _xw_body(x_ref, w_ref, xw_ref):
    xw_ref[...] = jnp.dot(x_ref[...], w_ref[...],
                          preferred_element_type=jnp.float32)


def _xw(x2, wcat):
    return pl.pallas_call(
        _xw_body,
        grid=(_N_BLKS,),
        in_specs=[
            pl.BlockSpec((_ROWS_BLK, D_FEAT), lambda i: (i, 0)),
            pl.BlockSpec((D_FEAT, DH2), lambda i: (0, 0)),
        ],
        out_specs=pl.BlockSpec((_ROWS_BLK, DH2), lambda i: (i, 0)),
        out_shape=jax.ShapeDtypeStruct((NPAD, DH2), jnp.float32),
    )(x2, wcat)


def _scale_body(xw_ref, deg0_ref, deg1_ref, y_ref):
    i = pl.program_id(0)
    deg = deg0_ref[...] + deg1_ref[...] + 1.0
    dis = lax.rsqrt(deg)
    rows = jax.lax.broadcasted_iota(jnp.int32, (_ROWS_BLK, 1), 0) + i * _ROWS_BLK
    y_ref[...] = jnp.where(rows < N_NODES, xw_ref[...] * dis[:, None], 0.0)


def _scale(xw, deg0, deg1):
    return pl.pallas_call(
        _scale_body,
        grid=(_N_BLKS,),
        in_specs=[
            pl.BlockSpec((_ROWS_BLK, DH2), lambda i: (i, 0)),
            pl.BlockSpec((_ROWS_BLK,), lambda i: (i,)),
            pl.BlockSpec((_ROWS_BLK,), lambda i: (i,)),
        ],
        out_specs=pl.BlockSpec((_ROWS_BLK, DH2), lambda i: (i, 0)),
        out_shape=jax.ShapeDtypeStruct((NPAD, DH2), jnp.float32),
    )(xw, deg0, deg1)


_FIN_BLK = 1024
_FIN_N = (NPAD // 2) // _FIN_BLK


def _fin_body(acc_ref, y_ref, dege_ref, dego_ref, m1_ref, m2_ref,
              bz_ref, bh_ref, wl_ref, bl_ref, oute_ref, outo_ref):
    p = acc_ref[0] + acc_ref[1] - y_ref[...]
    dise = lax.rsqrt(dege_ref[...] + 1.0)
    diso = lax.rsqrt(dego_ref[...] + 1.0)
    for half, dis, out_ref in ((0, dise, oute_ref), (1, diso, outo_ref)):
        t = p[:, half * DH2:(half + 1) * DH2] * dis[:, None]
        zin = jnp.dot(t, m1_ref[...], preferred_element_type=jnp.float32) + bz_ref[...]
        hin = jnp.dot(t, m2_ref[...], preferred_element_type=jnp.float32) + bh_ref[...]
        f = (1.0 - jax.nn.sigmoid(zin)) * jnp.tanh(hin)
        out_ref[...] = jnp.sum(f * wl_ref[...], axis=1) + bl_ref[0, 0]


def _final(acc_p, y_p, dege, dego, m1, m2, bz, bh, wlrow, blin):
    return pl.pallas_call(
        _fin_body,
        grid=(_FIN_N,),
        in_specs=[
            pl.BlockSpec((NC, _FIN_BLK, 2 * DH2), lambda i: (0, i, 0)),
            pl.BlockSpec((_FIN_BLK, 2 * DH2), lambda i: (i, 0)),
            pl.BlockSpec((_FIN_BLK,), lambda i: (i,)),
            pl.BlockSpec((_FIN_BLK,), lambda i: (i,)),
            pl.BlockSpec((DH2, D_HID), lambda i: (0, 0)),
            pl.BlockSpec((DH2, D_HID), lambda i: (0, 0)),
            pl.BlockSpec((1, D_HID), lambda i: (0, 0)),
            pl.BlockSpec((1, D_HID), lambda i: (0, 0)),
            pl.BlockSpec((1, D_HID), lambda i: (0, 0)),
            pl.BlockSpec((1, 1), lambda i: (0, 0)),
        ],
        out_specs=(pl.BlockSpec((_FIN_BLK,), lambda i: (i,)),
                   pl.BlockSpec((_FIN_BLK,), lambda i: (i,))),
        out_shape=(jax.ShapeDtypeStruct((NPAD // 2,), jnp.float32),
                   jax.ShapeDtypeStruct((NPAD // 2,), jnp.float32)),
    )(acc_p, y_p, dege, dego, m1, m2, bz, bh, wlrow, blin)


def kernel(x, edge_index, W_z, b_z, W_r, b_r, W_h, b_h, lz_W, lz_b,
           lr_W, lr_b, lh_W, lh_b, att, W_lin, b_lin):
    f32 = jnp.float32
    x2 = x[:, :, 0]
    wcat = jnp.concatenate([W_z, W_h], axis=1)

    src_flat = edge_index[0]
    dst_flat = edge_index[1]

    zeros_init = jnp.zeros((NW * ROWS_PER_TILE,), f32)
    ones_vals = jnp.ones((NW * EDGES_PER_TILE,), f32)

    xw = _xw(x2, wcat)
    deg_flat = _deg_kernel(dst_flat, zeros_init, ones_vals)
    deg0 = deg_flat[:NPAD]
    deg1 = deg_flat[NPAD:]
    y = _scale(xw, deg0, deg1)
    y_lin = y.reshape(NPAD * DH2)
    y_sc = y_lin.reshape(NPAD, DH2)
    acc = _agg_kernel(y_sc, src_flat, dst_flat)

    acc_p = acc.reshape(NC, NPAD // 2, 2 * DH2)
    y_p = y_lin.reshape(NPAD // 2, 2 * DH2)
    deg = deg0 + deg1
    dege = deg[0::2]
    dego = deg[1::2]

    zeros32 = jnp.zeros((D_HID, D_HID), f32)
    m1 = jnp.concatenate([lz_W[:D_HID], zeros32], axis=0)
    m2 = jnp.concatenate([zeros32, lh_W[:D_HID]], axis=0)
    bz = (b_z @ lz_W[:D_HID] + lz_b).reshape(1, D_HID)
    bh = (b_h @ lh_W[:D_HID] + lh_b).reshape(1, D_HID)
    wlrow = W_lin[:, 0].reshape(1, D_HID)
    blin = b_lin.reshape(1, 1)

    oute, outo = _final(acc_p, y_p, dege, dego, m1, m2, bz, bh, wlrow, blin)
    out = jnp.stack([oute, outo], axis=1).reshape(-1)
    return out[:N_NODES]


# speedup vs baseline: 1.5811x; 1.0125x over previous
import functools

import jax
import jax.numpy as jnp
from jax import lax
from jax.experimental import pallas as pl
from jax.experimental.pallas import tpu as pltpu
from jax.experimental.pallas import tpu_sc as plsc

N_NODES = 10000
D_FEAT = 256
D_HID = 32
DH2 = 2 * D_HID

NC = 2
NS = 16
NW = NC * NS
EDGES_PER_TILE = 5000
CHUNK = 640
NFULL = 7
TAIL = EDGES_PER_TILE - NFULL * CHUNK
NPAD = 10240
ROWS_PER_TILE = NPAD // NS

_SC_MESH = dict(core_axis_name="c", subcore_axis_name="s",
                num_cores=NC, num_subcores=NS)
_SC_PARAMS = pltpu.CompilerParams(use_tc_tiling_on_sc=False)


def _deg_body(dst_hbm, zeros_hbm, ones_hbm, out_hbm, idx_v, ones_v, acc_sh):
    cid = lax.axis_index("c")
    sid = lax.axis_index("s")
    wid = cid * NS + sid
    row0 = sid * ROWS_PER_TILE
    pltpu.sync_copy(zeros_hbm.at[pl.ds(wid * ROWS_PER_TILE, ROWS_PER_TILE)],
                    acc_sh.at[pl.ds(row0, ROWS_PER_TILE)])
    pltpu.sync_copy(ones_hbm.at[pl.ds(wid * EDGES_PER_TILE, EDGES_PER_TILE)],
                    ones_v)
    pltpu.sync_copy(dst_hbm.at[pl.ds(wid * EDGES_PER_TILE, EDGES_PER_TILE)],
                    idx_v)
    plsc.subcore_barrier()
    pltpu.sync_copy(ones_v, acc_sh.at[idx_v], add=True)
    plsc.subcore_barrier()
    pltpu.sync_copy(acc_sh.at[pl.ds(row0, ROWS_PER_TILE)],
                    out_hbm.at[pl.ds(cid * NPAD + row0, ROWS_PER_TILE)])


_deg_kernel = functools.partial(
    pl.kernel,
    out_type=jax.ShapeDtypeStruct((NC * NPAD,), jnp.float32),
    mesh=plsc.VectorSubcoreMesh(**_SC_MESH),
    scratch_types=[
        pltpu.VMEM((EDGES_PER_TILE,), jnp.int32),
        pltpu.VMEM((EDGES_PER_TILE,), jnp.float32),
        pltpu.VMEM_SHARED((NPAD,), jnp.float32),
    ],
)(_deg_body)


_SIZES = [CHUNK] * NFULL + [TAIL]
_NCH = NFULL + 1


def _agg_body(y_hbm, src_hbm, dst_hbm, out_hbm,
              s0_v, s1_v, dst_v, st_v, dt_v, r0_v, r1_v, acc_sh, sem):
    cid = lax.axis_index("c")
    sid = lax.axis_index("s")
    wid = cid * NS + sid
    row0 = sid * ROWS_PER_TILE
    ebase = wid * EDGES_PER_TILE
    sbufs = (s0_v, s1_v)
    rbufs = (r0_v, r1_v)

    def src_ref(j):
        return st_v if _SIZES[j] == TAIL else sbufs[j % 2]

    def load_src(j):
        pltpu.sync_copy(src_hbm.at[pl.ds(ebase + j * CHUNK, _SIZES[j])],
                        src_ref(j))

    def start_gather(j):
        rows = rbufs[j % 2]
        if _SIZES[j] != CHUNK:
            rows = rows.at[pl.ds(0, _SIZES[j])]
        return pltpu.async_copy(y_hbm.at[src_ref(j)], rows, sem)

    load_src(0)
    cp = start_gather(0)
    pltpu.sync_copy(y_hbm.at[pl.ds(row0, ROWS_PER_TILE)],
                    acc_sh.at[pl.ds(row0, ROWS_PER_TILE)])
    plsc.subcore_barrier()
    for j in range(_NCH):
        if j + 1 < _NCH:
            load_src(j + 1)
        dref = dt_v if _SIZES[j] == TAIL else dst_v
        pltpu.sync_copy(dst_hbm.at[pl.ds(ebase + j * CHUNK, _SIZES[j])], dref)
        cp.wait()
        if j + 1 < _NCH:
            cp = start_gather(j + 1)
        rows = rbufs[j % 2]
        if _SIZES[j] != CHUNK:
            rows = rows.at[pl.ds(0, _SIZES[j])]
        pltpu.sync_copy(rows, acc_sh.at[dref], add=True)
    plsc.subcore_barrier()
    pltpu.sync_copy(acc_sh.at[pl.ds(row0, ROWS_PER_TILE)],
                    out_hbm.at[cid, pl.ds(row0, ROWS_PER_TILE)])


_agg_kernel = functools.partial(
    pl.kernel,
    out_type=jax.ShapeDtypeStruct((NC, NPAD, DH2), jnp.float32),
    mesh=plsc.VectorSubcoreMesh(**_SC_MESH),
    compiler_params=_SC_PARAMS,
    scratch_types=[
        pltpu.VMEM((CHUNK,), jnp.int32),
        pltpu.VMEM((CHUNK,), jnp.int32),
        pltpu.VMEM((CHUNK,), jnp.int32),
        pltpu.VMEM((TAIL,), jnp.int32),
        pltpu.VMEM((TAIL,), jnp.int32),
        pltpu.VMEM((CHUNK, DH2), jnp.float32),
        pltpu.VMEM((CHUNK, DH2), jnp.float32),
        pltpu.VMEM_SHARED((NPAD, DH2), jnp.float32),
        pltpu.SemaphoreType.DMA,
    ],
)(_agg_body)


_ROWS_BLK = 1024
_N_BLKS = NPAD // _ROWS_BLK


def _xw_body(x_ref, w_ref, xw_ref):
    xw_ref[...] = jnp.dot(x_ref[...], w_ref[...],
                          preferred_element_type=jnp.float32)


def _xw(x2, wcat):
    return pl.pallas_call(
        _xw_body,
        grid=(_N_BLKS,),
        in_specs=[
            pl.BlockSpec((_ROWS_BLK, D_FEAT), lambda i: (i, 0)),
            pl.BlockSpec((D_FEAT, DH2), lambda i: (0, 0)),
        ],
        out_specs=pl.BlockSpec((_ROWS_BLK, DH2), lambda i: (i, 0)),
        out_shape=jax.ShapeDtypeStruct((NPAD, DH2), jnp.float32),
    )(x2, wcat)


def _scale_body(xw_ref, deg0_ref, deg1_ref, y_ref):
    i = pl.program_id(0)
    deg = deg0_ref[...] + deg1_ref[...] + 1.0
    dis = lax.rsqrt(deg)
    rows = jax.lax.broadcasted_iota(jnp.int32, (_ROWS_BLK, 1), 0) + i * _ROWS_BLK
    y_ref[...] = jnp.where(rows < N_NODES, xw_ref[...] * dis[:, None], 0.0)


def _scale(xw, deg0, deg1):
    return pl.pallas_call(
        _scale_body,
        grid=(_N_BLKS,),
        in_specs=[
            pl.BlockSpec((_ROWS_BLK, DH2), lambda i: (i, 0)),
            pl.BlockSpec((_ROWS_BLK,), lambda i: (i,)),
            pl.BlockSpec((_ROWS_BLK,), lambda i: (i,)),
        ],
        out_specs=pl.BlockSpec((_ROWS_BLK, DH2), lambda i: (i, 0)),
        out_shape=jax.ShapeDtypeStruct((NPAD, DH2), jnp.float32),
    )(xw, deg0, deg1)


_FIN_BLK = 1024
_FIN_N = (NPAD // 2) // _FIN_BLK


def _fin_body(acc_ref, y_ref, dege_ref, dego_ref, m1_ref, m2_ref,
              bz_ref, bh_ref, wl_ref, bl_ref, oute_ref, outo_ref):
    p = acc_ref[0] + acc_ref[1] - y_ref[...]
    dise = lax.rsqrt(dege_ref[...] + 1.0)
    diso = lax.rsqrt(dego_ref[...] + 1.0)
    for half, dis, out_ref in ((0, dise, oute_ref), (1, diso, outo_ref)):
        t = p[:, half * DH2:(half + 1) * DH2] * dis[:, None]
        zin = jnp.dot(t, m1_ref[...], preferred_element_type=jnp.float32) + bz_ref[...]
        hin = jnp.dot(t, m2_ref[...], preferred_element_type=jnp.float32) + bh_ref[...]
        f = (1.0 - jax.nn.sigmoid(zin)) * jnp.tanh(hin)
        out_ref[...] = jnp.sum(f * wl_ref[...], axis=1) + bl_ref[0, 0]


def _final(acc_p, y_p, dege, dego, m1, m2, bz, bh, wlrow, blin):
    return pl.pallas_call(
        _fin_body,
        grid=(_FIN_N,),
        in_specs=[
            pl.BlockSpec((NC, _FIN_BLK, 2 * DH2), lambda i: (0, i, 0)),
            pl.BlockSpec((_FIN_BLK, 2 * DH2), lambda i: (i, 0)),
            pl.BlockSpec((_FIN_BLK,), lambda i: (i,)),
            pl.BlockSpec((_FIN_BLK,), lambda i: (i,)),
            pl.BlockSpec((DH2, D_HID), lambda i: (0, 0)),
            pl.BlockSpec((DH2, D_HID), lambda i: (0, 0)),
            pl.BlockSpec((1, D_HID), lambda i: (0, 0)),
            pl.BlockSpec((1, D_HID), lambda i: (0, 0)),
            pl.BlockSpec((1, D_HID), lambda i: (0, 0)),
            pl.BlockSpec((1, 1), lambda i: (0, 0)),
        ],
        out_specs=(pl.BlockSpec((_FIN_BLK,), lambda i: (i,)),
                   pl.BlockSpec((_FIN_BLK,), lambda i: (i,))),
        out_shape=(jax.ShapeDtypeStruct((NPAD // 2,), jnp.float32),
                   jax.ShapeDtypeStruct((NPAD // 2,), jnp.float32)),
    )(acc_p, y_p, dege, dego, m1, m2, bz, bh, wlrow, blin)


def kernel(x, edge_index, W_z, b_z, W_r, b_r, W_h, b_h, lz_W, lz_b,
           lr_W, lr_b, lh_W, lh_b, att, W_lin, b_lin):
    f32 = jnp.float32
    x2 = x[:, :, 0]
    wcat = jnp.concatenate([W_z, W_h], axis=1)

    src_flat = edge_index[0]
    dst_flat = edge_index[1]

    zeros_init = jnp.zeros((NW * ROWS_PER_TILE,), f32)
    ones_vals = jnp.ones((NW * EDGES_PER_TILE,), f32)

    xw = _xw(x2, wcat)
    deg_flat = _deg_kernel(dst_flat, zeros_init, ones_vals)
    deg0 = deg_flat[:NPAD]
    deg1 = deg_flat[NPAD:]
    y = _scale(xw, deg0, deg1)
    y_lin = y.reshape(NPAD * DH2)
    y_sc = y_lin.reshape(NPAD, DH2)
    acc = _agg_kernel(y_sc, src_flat, dst_flat)

    acc_p = acc.reshape(NC, NPAD // 2, 2 * DH2)
    y_p = y_lin.reshape(NPAD // 2, 2 * DH2)
    deg = deg0 + deg1
    dege = deg[0::2]
    dego = deg[1::2]

    zeros32 = jnp.zeros((D_HID, D_HID), f32)
    m1 = jnp.concatenate([lz_W[:D_HID], zeros32], axis=0)
    m2 = jnp.concatenate([zeros32, lh_W[:D_HID]], axis=0)
    bz = (b_z @ lz_W[:D_HID] + lz_b).reshape(1, D_HID)
    bh = (b_h @ lh_W[:D_HID] + lh_b).reshape(1, D_HID)
    wlrow = W_lin[:, 0].reshape(1, D_HID)
    blin = b_lin.reshape(1, 1)

    oute, outo = _final(acc_p, y_p, dege, dego, m1, m2, bz, bh, wlrow, blin)
    out = jnp.stack([oute, outo], axis=1).reshape(-1)
    return out[:N_NODES]


# speedup vs baseline: 1.6269x; 1.0290x over previous
import functools

import jax
import jax.numpy as jnp
from jax import lax
from jax.experimental import pallas as pl
from jax.experimental.pallas import tpu as pltpu
from jax.experimental.pallas import tpu_sc as plsc

N_NODES = 10000
D_FEAT = 256
D_HID = 32
DH2 = 2 * D_HID

NC = 2
NS = 16
NW = NC * NS
EDGES_PER_TILE = 5000
CHUNK = 640
NFULL = 7
TAIL = EDGES_PER_TILE - NFULL * CHUNK
NPAD = 10240
ROWS_PER_TILE = NPAD // NS

_SC_MESH = dict(core_axis_name="c", subcore_axis_name="s",
                num_cores=NC, num_subcores=NS)
_SC_PARAMS = pltpu.CompilerParams(use_tc_tiling_on_sc=False)


def _deg_body(dst_hbm, zeros_hbm, ones_hbm, out_hbm, idx_v, ones_v, acc_sh):
    cid = lax.axis_index("c")
    sid = lax.axis_index("s")
    wid = cid * NS + sid
    row0 = sid * ROWS_PER_TILE
    pltpu.sync_copy(zeros_hbm.at[pl.ds(wid * ROWS_PER_TILE, ROWS_PER_TILE)],
                    acc_sh.at[pl.ds(row0, ROWS_PER_TILE)])
    pltpu.sync_copy(ones_hbm.at[pl.ds(wid * EDGES_PER_TILE, EDGES_PER_TILE)],
                    ones_v)
    pltpu.sync_copy(dst_hbm.at[pl.ds(wid * EDGES_PER_TILE, EDGES_PER_TILE)],
                    idx_v)
    plsc.subcore_barrier()
    pltpu.sync_copy(ones_v, acc_sh.at[idx_v], add=True)
    plsc.subcore_barrier()
    pltpu.sync_copy(acc_sh.at[pl.ds(row0, ROWS_PER_TILE)],
                    out_hbm.at[pl.ds(cid * NPAD + row0, ROWS_PER_TILE)])


_deg_kernel = functools.partial(
    pl.kernel,
    out_type=jax.ShapeDtypeStruct((NC * NPAD,), jnp.float32),
    mesh=plsc.VectorSubcoreMesh(**_SC_MESH),
    scratch_types=[
        pltpu.VMEM((EDGES_PER_TILE,), jnp.int32),
        pltpu.VMEM((EDGES_PER_TILE,), jnp.float32),
        pltpu.VMEM_SHARED((NPAD,), jnp.float32),
    ],
)(_deg_body)


_SIZES = [CHUNK] * NFULL + [TAIL]
_NCH = NFULL + 1


def _agg_body(y_hbm, src_hbm, dst_hbm, out_hbm,
              s0_v, s1_v, dst_v, st_v, dt_v, r0_v, r1_v, acc_sh, sem):
    cid = lax.axis_index("c")
    sid = lax.axis_index("s")
    wid = cid * NS + sid
    row0 = sid * ROWS_PER_TILE
    ebase = wid * EDGES_PER_TILE
    sbufs = (s0_v, s1_v)
    rbufs = (r0_v, r1_v)

    def src_ref(j):
        return st_v if _SIZES[j] == TAIL else sbufs[j % 2]

    def load_src(j):
        pltpu.sync_copy(src_hbm.at[pl.ds(ebase + j * CHUNK, _SIZES[j])],
                        src_ref(j))

    def start_gather(j):
        rows = rbufs[j % 2]
        if _SIZES[j] != CHUNK:
            rows = rows.at[pl.ds(0, _SIZES[j])]
        return pltpu.async_copy(y_hbm.at[src_ref(j)], rows, sem)

    load_src(0)
    cp = start_gather(0)
    pltpu.sync_copy(y_hbm.at[pl.ds(row0, ROWS_PER_TILE)],
                    acc_sh.at[pl.ds(row0, ROWS_PER_TILE)])
    plsc.subcore_barrier()
    for j in range(_NCH):
        if j + 1 < _NCH:
            load_src(j + 1)
        dref = dt_v if _SIZES[j] == TAIL else dst_v
        pltpu.sync_copy(dst_hbm.at[pl.ds(ebase + j * CHUNK, _SIZES[j])], dref)
        cp.wait()
        if j + 1 < _NCH:
            cp = start_gather(j + 1)
        rows = rbufs[j % 2]
        if _SIZES[j] != CHUNK:
            rows = rows.at[pl.ds(0, _SIZES[j])]
        pltpu.sync_copy(rows, acc_sh.at[dref], add=True)
    plsc.subcore_barrier()
    pltpu.sync_copy(acc_sh.at[pl.ds(row0, ROWS_PER_TILE)],
                    out_hbm.at[cid, pl.ds(row0, ROWS_PER_TILE)])


_agg_kernel = functools.partial(
    pl.kernel,
    out_type=jax.ShapeDtypeStruct((NC, NPAD, DH2), jnp.float32),
    mesh=plsc.VectorSubcoreMesh(**_SC_MESH),
    compiler_params=_SC_PARAMS,
    scratch_types=[
        pltpu.VMEM((CHUNK,), jnp.int32),
        pltpu.VMEM((CHUNK,), jnp.int32),
        pltpu.VMEM((CHUNK,), jnp.int32),
        pltpu.VMEM((TAIL,), jnp.int32),
        pltpu.VMEM((TAIL,), jnp.int32),
        pltpu.VMEM((CHUNK, DH2), jnp.float32),
        pltpu.VMEM((CHUNK, DH2), jnp.float32),
        pltpu.VMEM_SHARED((NPAD, DH2), jnp.float32),
        pltpu.SemaphoreType.DMA,
    ],
)(_agg_body)


_ROWS_BLK = 2048
_N_BLKS = NPAD // _ROWS_BLK


def _xw_body(x_ref, w_ref, xw_ref):
    xw_ref[...] = jnp.dot(x_ref[...], w_ref[...],
                          preferred_element_type=jnp.float32)


def _xw(x2, wcat):
    return pl.pallas_call(
        _xw_body,
        grid=(_N_BLKS,),
        in_specs=[
            pl.BlockSpec((_ROWS_BLK, D_FEAT), lambda i: (i, 0)),
            pl.BlockSpec((D_FEAT, DH2), lambda i: (0, 0)),
        ],
        out_specs=pl.BlockSpec((_ROWS_BLK, DH2), lambda i: (i, 0)),
        out_shape=jax.ShapeDtypeStruct((NPAD, DH2), jnp.float32),
    )(x2, wcat)


def _scale_body(xw_ref, deg0_ref, deg1_ref, y_ref):
    i = pl.program_id(0)
    deg = deg0_ref[...] + deg1_ref[...] + 1.0
    dis = lax.rsqrt(deg)
    rows = jax.lax.broadcasted_iota(jnp.int32, (_ROWS_BLK, 1), 0) + i * _ROWS_BLK
    y_ref[...] = jnp.where(rows < N_NODES, xw_ref[...] * dis[:, None], 0.0)


def _scale(xw, deg0, deg1):
    return pl.pallas_call(
        _scale_body,
        grid=(_N_BLKS,),
        in_specs=[
            pl.BlockSpec((_ROWS_BLK, DH2), lambda i: (i, 0)),
            pl.BlockSpec((_ROWS_BLK,), lambda i: (i,)),
            pl.BlockSpec((_ROWS_BLK,), lambda i: (i,)),
        ],
        out_specs=pl.BlockSpec((_ROWS_BLK, DH2), lambda i: (i, 0)),
        out_shape=jax.ShapeDtypeStruct((NPAD, DH2), jnp.float32),
    )(xw, deg0, deg1)


_FIN_BLK = 1024
_FIN_N = (NPAD // 2) // _FIN_BLK


def _fin_body(acc_ref, y_ref, dege_ref, dego_ref, m1_ref, m2_ref,
              bz_ref, bh_ref, wl_ref, bl_ref, oute_ref, outo_ref):
    p = acc_ref[0] + acc_ref[1] - y_ref[...]
    dise = lax.rsqrt(dege_ref[...] + 1.0)
    diso = lax.rsqrt(dego_ref[...] + 1.0)
    for half, dis, out_ref in ((0, dise, oute_ref), (1, diso, outo_ref)):
        t = p[:, half * DH2:(half + 1) * DH2] * dis[:, None]
        zin = jnp.dot(t, m1_ref[...], preferred_element_type=jnp.float32) + bz_ref[...]
        hin = jnp.dot(t, m2_ref[...], preferred_element_type=jnp.float32) + bh_ref[...]
        f = (1.0 - jax.nn.sigmoid(zin)) * jnp.tanh(hin)
        out_ref[...] = jnp.sum(f * wl_ref[...], axis=1) + bl_ref[0, 0]


def _final(acc_p, y_p, dege, dego, m1, m2, bz, bh, wlrow, blin):
    return pl.pallas_call(
        _fin_body,
        grid=(_FIN_N,),
        in_specs=[
            pl.BlockSpec((NC, _FIN_BLK, 2 * DH2), lambda i: (0, i, 0)),
            pl.BlockSpec((_FIN_BLK, 2 * DH2), lambda i: (i, 0)),
            pl.BlockSpec((_FIN_BLK,), lambda i: (i,)),
            pl.BlockSpec((_FIN_BLK,), lambda i: (i,)),
            pl.BlockSpec((DH2, D_HID), lambda i: (0, 0)),
            pl.BlockSpec((DH2, D_HID), lambda i: (0, 0)),
            pl.BlockSpec((1, D_HID), lambda i: (0, 0)),
            pl.BlockSpec((1, D_HID), lambda i: (0, 0)),
            pl.BlockSpec((1, D_HID), lambda i: (0, 0)),
            pl.BlockSpec((1, 1), lambda i: (0, 0)),
        ],
        out_specs=(pl.BlockSpec((_FIN_BLK,), lambda i: (i,)),
                   pl.BlockSpec((_FIN_BLK,), lambda i: (i,))),
        out_shape=(jax.ShapeDtypeStruct((NPAD // 2,), jnp.float32),
                   jax.ShapeDtypeStruct((NPAD // 2,), jnp.float32)),
    )(acc_p, y_p, dege, dego, m1, m2, bz, bh, wlrow, blin)


def kernel(x, edge_index, W_z, b_z, W_r, b_r, W_h, b_h, lz_W, lz_b,
           lr_W, lr_b, lh_W, lh_b, att, W_lin, b_lin):
    f32 = jnp.float32
    x2 = x[:, :, 0]
    wcat = jnp.concatenate([W_z, W_h], axis=1)

    src_flat = edge_index[0]
    dst_flat = edge_index[1]

    zeros_init = jnp.zeros((NW * ROWS_PER_TILE,), f32)
    ones_vals = jnp.ones((NW * EDGES_PER_TILE,), f32)

    xw = _xw(x2, wcat)
    deg_flat = _deg_kernel(dst_flat, zeros_init, ones_vals)
    deg0 = deg_flat[:NPAD]
    deg1 = deg_flat[NPAD:]
    y = _scale(xw, deg0, deg1)
    y_lin = y.reshape(NPAD * DH2)
    y_sc = y_lin.reshape(NPAD, DH2)
    acc = _agg_kernel(y_sc, src_flat, dst_flat)

    acc_p = acc.reshape(NC, NPAD // 2, 2 * DH2)
    y_p = y_lin.reshape(NPAD // 2, 2 * DH2)
    deg = deg0 + deg1
    dege = deg[0::2]
    dego = deg[1::2]

    zeros32 = jnp.zeros((D_HID, D_HID), f32)
    m1 = jnp.concatenate([lz_W[:D_HID], zeros32], axis=0)
    m2 = jnp.concatenate([zeros32, lh_W[:D_HID]], axis=0)
    bz = (b_z @ lz_W[:D_HID] + lz_b).reshape(1, D_HID)
    bh = (b_h @ lh_W[:D_HID] + lh_b).reshape(1, D_HID)
    wlrow = W_lin[:, 0].reshape(1, D_HID)
    blin = b_lin.reshape(1, 1)

    oute, outo = _final(acc_p, y_p, dege, dego, m1, m2, bz, bh, wlrow, blin)
    out = jnp.stack([oute, outo], axis=1).reshape(-1)
    return out[:N_NODES]
